# baseline (speedup 1.0000x reference)
import functools
import math

import jax
import jax.numpy as jnp
from jax import lax
from jax.experimental import pallas as pl
from jax.experimental.pallas import tpu as pltpu
from jax.experimental.pallas import tpu_sc as plsc

_EMBED = 32
_SCALE = math.sqrt(float(_EMBED))
_NUM_WORKERS = 32
_CHUNK = 512
_PACK_C = 2048


def _pack_body(in_ref, out_ref):
    a = in_ref[...] * _SCALE
    for j in range(4):
        cols = a[:, 512 * j:512 * (j + 1)]
        stacked = jnp.concatenate(
            [cols[:, 0:128], cols[:, 128:256],
             cols[:, 256:384], cols[:, 384:512]], axis=0)
        out_ref[128 * j:128 * (j + 1), :] = stacked.T


def _unpack_body(in_ref, out_ref):
    t = in_ref[...].T
    out_ref[0, :, 0:128] = t[0:32, :]
    out_ref[0, :, 128:256] = t[32:64, :]
    out_ref[0, :, 256:384] = t[64:96, :]
    out_ref[0, :, 384:512] = t[96:128, :]


def kernel(x, lut):
    batch, seq = x.shape
    n = batch * seq
    vocab, embed = lut.shape
    idx = jnp.transpose(x).reshape(n).astype(jnp.int32)

    lut_t = jnp.transpose(lut)
    grid = (vocab + _PACK_C - 1) // _PACK_C
    vocab_pad = grid * _PACK_C
    lut_packed = pl.pallas_call(
        _pack_body,
        grid=(grid,),
        in_specs=[pl.BlockSpec((embed, _PACK_C), lambda i: (0, i))],
        out_specs=pl.BlockSpec((_PACK_C * embed // 128, 128),
                               lambda i: (i, 0)),
        out_shape=jax.ShapeDtypeStruct((vocab_pad * embed // 128, 128),
                                       jnp.float32),
        compiler_params=pltpu.CompilerParams(
            dimension_semantics=("parallel",)),
    )(lut_t)
    lut_rows = lut_packed.reshape(vocab_pad, embed)

    per_w = n // _NUM_WORKERS
    n_chunks = per_w // _CHUNK
    assert n_chunks % 2 == 0 and n_chunks * _CHUNK == per_w

    mesh = plsc.VectorSubcoreMesh(core_axis_name="c", subcore_axis_name="s")

    @functools.partial(
        pl.kernel,
        mesh=mesh,
        out_type=jax.ShapeDtypeStruct((n, embed), jnp.float32),
        scratch_types=[
            pltpu.VMEM((_CHUNK,), jnp.int32),
            pltpu.VMEM((_CHUNK,), jnp.int32),
            pltpu.VMEM((_CHUNK,), jnp.int32),
            pltpu.VMEM((_CHUNK, embed), jnp.float32),
            pltpu.VMEM((_CHUNK, embed), jnp.float32),
            pltpu.SemaphoreType.DMA,
            pltpu.SemaphoreType.DMA,
            pltpu.SemaphoreType.DMA,
            pltpu.SemaphoreType.DMA,
        ],
        compiler_params=pltpu.CompilerParams(use_tc_tiling_on_sc=False,
                                             needs_layout_passes=False),
    )
    def sc_gather(idx_hbm, table_hbm, out_hbm,
                  idx_raw, idx0, idx1, rows0, rows1,
                  gsem0, gsem1, ssem0, ssem1):
        wid = lax.axis_index("s") * 2 + lax.axis_index("c")
        base = wid * per_w
        iota16 = lax.iota(jnp.int32, 16)

        def start(c, idx_b, rows_b, gsem_b):
            off = base + c * _CHUNK
            pltpu.sync_copy(idx_hbm.at[pl.ds(off, _CHUNK)], idx_raw)

            @pl.loop(0, _CHUNK, step=16)
            def _(m):
                v = idx_raw.at[pl.ds(m, 16)][...]
                rho = (((v >> 11) << 11) + (((v >> 9) & 3) << 9)
                       + ((v & 127) << 2) + ((v >> 7) & 3))
                pos = ((m & 127) + iota16) * 4 + ((m >> 7) & 3)
                plsc.store_scatter(idx_b, [pos], rho)

            pltpu.async_copy(table_hbm.at[idx_b], rows_b, gsem_b)

        def gather_wait(idx_b, rows_b, gsem_b):
            pltpu.make_async_copy(table_hbm.at[idx_b], rows_b, gsem_b).wait()

        def store_wait(rows_b, ssem_b):
            pltpu.make_async_copy(
                rows_b, out_hbm.at[pl.ds(0, _CHUNK)], ssem_b).wait()

        start(0, idx0, rows0, gsem0)

        @pl.loop(0, n_chunks, step=2)
        def _(c):
            @pl.when(c > 0)
            def _():
                store_wait(rows1, ssem1)

            start(c + 1, idx1, rows1, gsem1)

            gather_wait(idx0, rows0, gsem0)
            pltpu.async_copy(
                rows0, out_hbm.at[pl.ds(base + c * _CHUNK, _CHUNK)], ssem0)

            @pl.when(c + 2 < n_chunks)
            def _():
                store_wait(rows0, ssem0)
                start(c + 2, idx0, rows0, gsem0)

            gather_wait(idx1, rows1, gsem1)
            pltpu.async_copy(
                rows1, out_hbm.at[pl.ds(base + (c + 1) * _CHUNK, _CHUNK)],
                ssem1)

        store_wait(rows0, ssem0)
        store_wait(rows1, ssem1)

    gathered = sc_gather(idx, lut_rows)

    g128 = gathered.reshape(n // 4, 128)
    out_t = pl.pallas_call(
        _unpack_body,
        grid=(seq, batch // 512),
        in_specs=[pl.BlockSpec((128, 128), lambda s, k: (8 * s + k, 0))],
        out_specs=pl.BlockSpec((1, embed, 512), lambda s, k: (s, 0, k)),
        out_shape=jax.ShapeDtypeStruct((seq, embed, batch), jnp.float32),
        compiler_params=pltpu.CompilerParams(
            dimension_semantics=("parallel", "parallel")),
    )(g128)
    return jnp.transpose(out_t, (2, 0, 1))


# speedup vs baseline: 3.2514x; 3.2514x over previous
import functools
import math

import jax
import jax.numpy as jnp
from jax import lax
from jax.experimental import pallas as pl
from jax.experimental.pallas import tpu as pltpu
from jax.experimental.pallas import tpu_sc as plsc

_EMBED = 32
_SCALE = math.sqrt(float(_EMBED))
_NUM_WORKERS = 32
_CHUNK = 1024
_PACK_C = 8192


def _pack_body(in_ref, out_ref):
    a = in_ref[...] * _SCALE
    for j in range(_PACK_C // 512):
        cols = a[:, 512 * j:512 * (j + 1)]
        stacked = jnp.concatenate(
            [cols[:, 0:128], cols[:, 128:256],
             cols[:, 256:384], cols[:, 384:512]], axis=0)
        out_ref[128 * j:128 * (j + 1), :] = stacked.T


def _unpack_body(in_ref, out_ref):
    for k in range(8):
        t = in_ref[128 * k:128 * (k + 1), :].T
        for q in range(4):
            out_ref[0, :, 512 * k + 128 * q:512 * k + 128 * (q + 1)] = (
                t[32 * q:32 * (q + 1), :])


def kernel(x, lut):
    batch, seq = x.shape
    n = batch * seq
    vocab, embed = lut.shape
    idx = jnp.transpose(x).reshape(n).astype(jnp.int32)

    lut_t = jnp.transpose(lut)
    grid = (vocab + _PACK_C - 1) // _PACK_C
    vocab_pad = grid * _PACK_C
    lut_packed = pl.pallas_call(
        _pack_body,
        grid=(grid,),
        in_specs=[pl.BlockSpec((embed, _PACK_C), lambda i: (0, i))],
        out_specs=pl.BlockSpec((_PACK_C * embed // 128, 128),
                               lambda i: (i, 0)),
        out_shape=jax.ShapeDtypeStruct((vocab_pad * embed // 128, 128),
                                       jnp.float32),
        compiler_params=pltpu.CompilerParams(
            dimension_semantics=("parallel",)),
    )(lut_t)
    lut_rows = lut_packed.reshape(vocab_pad, embed)

    per_w = n // _NUM_WORKERS
    n_chunks = per_w // _CHUNK
    assert n_chunks * _CHUNK == per_w and n_chunks % 2 == 1

    mesh = plsc.VectorSubcoreMesh(core_axis_name="c", subcore_axis_name="s")

    @functools.partial(
        pl.kernel,
        mesh=mesh,
        out_type=jax.ShapeDtypeStruct((n, embed), jnp.float32),
        scratch_types=[
            pltpu.VMEM((_CHUNK,), jnp.int32),
            pltpu.VMEM((_CHUNK,), jnp.int32),
            pltpu.VMEM((_CHUNK,), jnp.int32),
            pltpu.VMEM((_CHUNK, embed), jnp.float32),
            pltpu.VMEM((_CHUNK, embed), jnp.float32),
            pltpu.SemaphoreType.DMA,
            pltpu.SemaphoreType.DMA,
            pltpu.SemaphoreType.DMA,
            pltpu.SemaphoreType.DMA,
        ],
        compiler_params=pltpu.CompilerParams(use_tc_tiling_on_sc=False,
                                             needs_layout_passes=False),
    )
    def sc_gather(idx_hbm, table_hbm, out_hbm,
                  idx_raw, idx0, idx1, rows0, rows1,
                  gsem0, gsem1, ssem0, ssem1):
        wid = lax.axis_index("s") * 2 + lax.axis_index("c")
        base = wid * per_w
        iota16 = lax.iota(jnp.int32, 16)

        def start(c, idx_b, rows_b, gsem_b):
            off = base + c * _CHUNK
            pltpu.sync_copy(idx_hbm.at[pl.ds(off, _CHUNK)], idx_raw)

            @pl.loop(0, _CHUNK, step=16)
            def _(m):
                v = idx_raw.at[pl.ds(m, 16)][...]
                rho = ((v & ~511) + ((v & 127) << 2) + ((v >> 7) & 3))
                pos = ((m & ~511) + ((m & 127) + iota16) * 4
                       + ((m >> 7) & 3))
                plsc.store_scatter(idx_b, [pos], rho)

            pltpu.async_copy(table_hbm.at[idx_b], rows_b, gsem_b)

        def gather_wait(idx_b, rows_b, gsem_b):
            pltpu.make_async_copy(table_hbm.at[idx_b], rows_b, gsem_b).wait()

        def store_wait(rows_b, ssem_b):
            pltpu.make_async_copy(
                rows_b, out_hbm.at[pl.ds(0, _CHUNK)], ssem_b).wait()

        start(0, idx0, rows0, gsem0)

        @pl.loop(0, n_chunks - 1, step=2)
        def _(c):
            @pl.when(c > 0)
            def _():
                store_wait(rows1, ssem1)

            start(c + 1, idx1, rows1, gsem1)

            gather_wait(idx0, rows0, gsem0)
            pltpu.async_copy(
                rows0, out_hbm.at[pl.ds(base + c * _CHUNK, _CHUNK)], ssem0)

            @pl.when(c + 2 < n_chunks)
            def _():
                store_wait(rows0, ssem0)
                start(c + 2, idx0, rows0, gsem0)

            gather_wait(idx1, rows1, gsem1)
            pltpu.async_copy(
                rows1, out_hbm.at[pl.ds(base + (c + 1) * _CHUNK, _CHUNK)],
                ssem1)

        gather_wait(idx0, rows0, gsem0)
        pltpu.async_copy(
            rows0,
            out_hbm.at[pl.ds(base + (n_chunks - 1) * _CHUNK, _CHUNK)], ssem0)
        store_wait(rows1, ssem1)
        store_wait(rows0, ssem0)

    gathered = sc_gather(idx, lut_rows)

    g128 = gathered.reshape(n // 4, 128)
    out_t = pl.pallas_call(
        _unpack_body,
        grid=(seq,),
        in_specs=[pl.BlockSpec((batch // 4, 128), lambda s: (s, 0))],
        out_specs=pl.BlockSpec((1, embed, batch), lambda s: (s, 0, 0)),
        out_shape=jax.ShapeDtypeStruct((seq, embed, batch), jnp.float32),
        compiler_params=pltpu.CompilerParams(
            dimension_semantics=("parallel",)),
    )(g128)
    return jnp.transpose(out_t, (2, 0, 1))


# speedup vs baseline: 4.4995x; 1.3838x over previous
import functools
import math

import jax
import jax.numpy as jnp
from jax import lax
from jax.experimental import pallas as pl
from jax.experimental.pallas import tpu as pltpu
from jax.experimental.pallas import tpu_sc as plsc

_EMBED = 32
_SCALE = math.sqrt(float(_EMBED))
_NUM_WORKERS = 32
_CHUNK = 1024
_PACK_C = 16384


def _pack_body(in_ref, out_ref):
    a = in_ref[...] * _SCALE
    for j in range(_PACK_C // 512):
        cols = a[:, 512 * j:512 * (j + 1)]
        stacked = jnp.concatenate(
            [cols[:, 0:128], cols[:, 128:256],
             cols[:, 256:384], cols[:, 384:512]], axis=0)
        out_ref[128 * j:128 * (j + 1), :] = stacked.T


def _unpack_body(in_ref, out_ref):
    planes = out_ref.shape[0]
    for p in range(planes):
        for k in range(8):
            r = 1024 * p + 128 * k
            t = in_ref[r:r + 128, :].T
            for q in range(4):
                out_ref[p, :, 512 * k + 128 * q:512 * k + 128 * (q + 1)] = (
                    t[32 * q:32 * (q + 1), :])


def kernel(x, lut):
    batch, seq = x.shape
    n = batch * seq
    vocab, embed = lut.shape
    idx = jnp.transpose(x).reshape(n).astype(jnp.int32)

    lut_t = jnp.transpose(lut)
    grid = (vocab + _PACK_C - 1) // _PACK_C
    vocab_pad = grid * _PACK_C
    lut_packed = pl.pallas_call(
        _pack_body,
        grid=(grid,),
        in_specs=[pl.BlockSpec((embed, _PACK_C), lambda i: (0, i))],
        out_specs=pl.BlockSpec((_PACK_C * embed // 128, 128),
                               lambda i: (i, 0)),
        out_shape=jax.ShapeDtypeStruct((vocab_pad * embed // 128, 128),
                                       jnp.float32),
        compiler_params=pltpu.CompilerParams(
            dimension_semantics=("parallel",)),
    )(lut_t)
    lut_rows = lut_packed.reshape(vocab_pad, embed)

    per_w = n // _NUM_WORKERS
    n_chunks = per_w // _CHUNK
    assert n_chunks * _CHUNK == per_w and n_chunks % 2 == 1

    mesh = plsc.VectorSubcoreMesh(core_axis_name="c", subcore_axis_name="s")

    @functools.partial(
        pl.kernel,
        mesh=mesh,
        out_type=jax.ShapeDtypeStruct((n, embed), jnp.float32),
        scratch_types=[
            pltpu.VMEM((_CHUNK,), jnp.int32),
            pltpu.VMEM((_CHUNK,), jnp.int32),
            pltpu.VMEM((_CHUNK,), jnp.int32),
            pltpu.VMEM((_CHUNK, embed), jnp.float32),
            pltpu.VMEM((_CHUNK, embed), jnp.float32),
            pltpu.SemaphoreType.DMA,
            pltpu.SemaphoreType.DMA,
            pltpu.SemaphoreType.DMA,
            pltpu.SemaphoreType.DMA,
        ],
        compiler_params=pltpu.CompilerParams(use_tc_tiling_on_sc=False,
                                             needs_layout_passes=False),
    )
    def sc_gather(idx_hbm, table_hbm, out_hbm,
                  idx_raw, idx0, idx1, rows0, rows1,
                  gsem0, gsem1, ssem0, ssem1):
        wid = lax.axis_index("s") * 2 + lax.axis_index("c")
        base = wid * per_w
        iota16 = lax.iota(jnp.int32, 16)

        def start(c, idx_b, rows_b, gsem_b):
            off = base + c * _CHUNK
            pltpu.sync_copy(idx_hbm.at[pl.ds(off, _CHUNK)], idx_raw)

            @pl.loop(0, _CHUNK, step=16)
            def _(m):
                v = idx_raw.at[pl.ds(m, 16)][...]
                rho = ((v & ~511) + ((v & 127) << 2) + ((v >> 7) & 3))
                pos = ((m & ~511) + ((m & 127) + iota16) * 4
                       + ((m >> 7) & 3))
                plsc.store_scatter(idx_b, [pos], rho)

            pltpu.async_copy(table_hbm.at[idx_b], rows_b, gsem_b)

        def gather_wait(idx_b, rows_b, gsem_b):
            pltpu.make_async_copy(table_hbm.at[idx_b], rows_b, gsem_b).wait()

        def store_wait(rows_b, ssem_b):
            pltpu.make_async_copy(
                rows_b, out_hbm.at[pl.ds(0, _CHUNK)], ssem_b).wait()

        start(0, idx0, rows0, gsem0)

        @pl.loop(0, n_chunks - 1, step=2)
        def _(c):
            @pl.when(c > 0)
            def _():
                store_wait(rows1, ssem1)

            start(c + 1, idx1, rows1, gsem1)

            gather_wait(idx0, rows0, gsem0)
            pltpu.async_copy(
                rows0, out_hbm.at[pl.ds(base + c * _CHUNK, _CHUNK)], ssem0)

            @pl.when(c + 2 < n_chunks)
            def _():
                store_wait(rows0, ssem0)
                start(c + 2, idx0, rows0, gsem0)

            gather_wait(idx1, rows1, gsem1)
            pltpu.async_copy(
                rows1, out_hbm.at[pl.ds(base + (c + 1) * _CHUNK, _CHUNK)],
                ssem1)

        gather_wait(idx0, rows0, gsem0)
        pltpu.async_copy(
            rows0,
            out_hbm.at[pl.ds(base + (n_chunks - 1) * _CHUNK, _CHUNK)], ssem0)
        store_wait(rows1, ssem1)
        store_wait(rows0, ssem0)

    gathered = sc_gather(idx, lut_rows)

    g128 = gathered.reshape(n // 4, 128)
    planes_per_step = 4
    out_t = pl.pallas_call(
        _unpack_body,
        grid=(seq // planes_per_step,),
        in_specs=[pl.BlockSpec((planes_per_step * batch // 4, 128),
                               lambda s: (s, 0))],
        out_specs=pl.BlockSpec((planes_per_step, embed, batch),
                               lambda s: (s, 0, 0)),
        out_shape=jax.ShapeDtypeStruct((seq, embed, batch), jnp.float32),
        compiler_params=pltpu.CompilerParams(
            dimension_semantics=("parallel",)),
    )(g128)
    return jnp.transpose(out_t, (2, 0, 1))


# speedup vs baseline: 4.6775x; 1.0396x over previous
import functools
import math

import jax
import jax.numpy as jnp
from jax import lax
from jax.experimental import pallas as pl
from jax.experimental.pallas import tpu as pltpu
from jax.experimental.pallas import tpu_sc as plsc

_EMBED = 32
_SCALE = math.sqrt(float(_EMBED))
_NUM_WORKERS = 32
_CHUNK = 512
_PACK_C = 16384
_SLICES = 5
_PLANES_PER_STEP = 4


def _pack_body(in_ref, out_ref):
    a = in_ref[...] * _SCALE
    for j in range(_PACK_C // 512):
        cols = a[:, 512 * j:512 * (j + 1)]
        stacked = jnp.concatenate(
            [cols[:, 0:128], cols[:, 128:256],
             cols[:, 256:384], cols[:, 384:512]], axis=0)
        out_ref[128 * j:128 * (j + 1), :] = stacked.T


def _unpack_first_body(in_ref, out_ref):
    _unpack_planes(in_ref, out_ref)


def _unpack_body(in_ref, prev_ref, out_ref):
    del prev_ref
    _unpack_planes(in_ref, out_ref)


def _unpack_planes(in_ref, out_ref):
    for p in range(_PLANES_PER_STEP):
        for k in range(8):
            r = 1024 * p + 128 * k
            t = in_ref[r:r + 128, :].T
            for q in range(4):
                out_ref[p, :, 512 * k + 128 * q:512 * k + 128 * (q + 1)] = (
                    t[32 * q:32 * (q + 1), :])


def kernel(x, lut):
    batch, seq = x.shape
    n = batch * seq
    vocab, embed = lut.shape
    idx = jnp.transpose(x).reshape(n).astype(jnp.int32)

    lut_t = jnp.transpose(lut)
    grid = (vocab + _PACK_C - 1) // _PACK_C
    vocab_pad = grid * _PACK_C
    lut_packed = pl.pallas_call(
        _pack_body,
        grid=(grid,),
        in_specs=[pl.BlockSpec((embed, _PACK_C), lambda i: (0, i))],
        out_specs=pl.BlockSpec((_PACK_C * embed // 128, 128),
                               lambda i: (i, 0)),
        out_shape=jax.ShapeDtypeStruct((vocab_pad * embed // 128, 128),
                                       jnp.float32),
        compiler_params=pltpu.CompilerParams(
            dimension_semantics=("parallel",)),
    )(lut_t)
    lut_rows = lut_packed.reshape(vocab_pad, embed)

    n_sl = n // _SLICES
    seq_sl = seq // _SLICES
    per_w = n_sl // _NUM_WORKERS
    n_chunks = per_w // _CHUNK
    assert n_chunks * _CHUNK == per_w

    mesh = plsc.VectorSubcoreMesh(core_axis_name="c", subcore_axis_name="s")

    @functools.partial(
        pl.kernel,
        mesh=mesh,
        out_type=jax.ShapeDtypeStruct((n_sl, embed), jnp.float32),
        scratch_types=[
            pltpu.VMEM((_CHUNK,), jnp.int32),
            pltpu.VMEM((_CHUNK,), jnp.int32),
            pltpu.VMEM((_CHUNK,), jnp.int32),
            pltpu.VMEM((_CHUNK, embed), jnp.float32),
            pltpu.VMEM((_CHUNK, embed), jnp.float32),
            pltpu.SemaphoreType.DMA,
            pltpu.SemaphoreType.DMA,
            pltpu.SemaphoreType.DMA,
            pltpu.SemaphoreType.DMA,
        ],
        compiler_params=pltpu.CompilerParams(use_tc_tiling_on_sc=False,
                                             needs_layout_passes=False),
    )
    def sc_gather(idx_hbm, table_hbm, out_hbm,
                  idx_raw, idx0, idx1, rows0, rows1,
                  gsem0, gsem1, ssem0, ssem1):
        wid = lax.axis_index("s") * 2 + lax.axis_index("c")
        base = wid * per_w
        iota16 = lax.iota(jnp.int32, 16)

        def start(c, idx_b, rows_b, gsem_b):
            off = base + c * _CHUNK
            pltpu.sync_copy(idx_hbm.at[pl.ds(off, _CHUNK)], idx_raw)

            @pl.loop(0, _CHUNK, step=16)
            def _(m):
                v = idx_raw.at[pl.ds(m, 16)][...]
                rho = ((v & ~511) + ((v & 127) << 2) + ((v >> 7) & 3))
                pos = ((m & ~511) + ((m & 127) + iota16) * 4
                       + ((m >> 7) & 3))
                plsc.store_scatter(idx_b, [pos], rho)

            pltpu.async_copy(table_hbm.at[idx_b], rows_b, gsem_b)

        def gather_wait(idx_b, rows_b, gsem_b):
            pltpu.make_async_copy(table_hbm.at[idx_b], rows_b, gsem_b).wait()

        def store_wait(rows_b, ssem_b):
            pltpu.make_async_copy(
                rows_b, out_hbm.at[pl.ds(0, _CHUNK)], ssem_b).wait()

        start(0, idx0, rows0, gsem0)

        @pl.loop(0, (n_chunks // 2) * 2, step=2)
        def _(c):
            @pl.when(c > 0)
            def _():
                store_wait(rows1, ssem1)

            start(c + 1, idx1, rows1, gsem1)

            gather_wait(idx0, rows0, gsem0)
            pltpu.async_copy(
                rows0, out_hbm.at[pl.ds(base + c * _CHUNK, _CHUNK)], ssem0)

            @pl.when(c + 2 < n_chunks)
            def _():
                store_wait(rows0, ssem0)
                start(c + 2, idx0, rows0, gsem0)

            gather_wait(idx1, rows1, gsem1)
            pltpu.async_copy(
                rows1, out_hbm.at[pl.ds(base + (c + 1) * _CHUNK, _CHUNK)],
                ssem1)

        if n_chunks % 2 == 1:
            gather_wait(idx0, rows0, gsem0)
            pltpu.async_copy(
                rows0,
                out_hbm.at[pl.ds(base + (n_chunks - 1) * _CHUNK, _CHUNK)],
                ssem0)
        store_wait(rows1, ssem1)
        store_wait(rows0, ssem0)

    out_t = None
    for k in range(_SLICES):
        idx_k = lax.slice(idx, (k * n_sl,), ((k + 1) * n_sl,))
        gathered = sc_gather(idx_k, lut_rows)
        g128 = gathered.reshape(n_sl // 4, 128)
        out_map = lambda s, kk=k: (kk * (seq_sl // _PLANES_PER_STEP) + s,
                                   0, 0)
        in_spec = pl.BlockSpec((_PLANES_PER_STEP * batch // 4, 128),
                               lambda s: (s, 0))
        out_spec = pl.BlockSpec((_PLANES_PER_STEP, embed, batch), out_map)
        out_shape = jax.ShapeDtypeStruct((seq, embed, batch), jnp.float32)
        cp = pltpu.CompilerParams(dimension_semantics=("arbitrary",))
        if out_t is None:
            out_t = pl.pallas_call(
                _unpack_first_body,
                grid=(seq_sl // _PLANES_PER_STEP,),
                in_specs=[in_spec],
                out_specs=out_spec,
                out_shape=out_shape,
                compiler_params=cp,
            )(g128)
        else:
            out_t = pl.pallas_call(
                _unpack_body,
                grid=(seq_sl // _PLANES_PER_STEP,),
                in_specs=[in_spec,
                          pl.BlockSpec(memory_space=pl.ANY)],
                out_specs=out_spec,
                out_shape=out_shape,
                input_output_aliases={1: 0},
                compiler_params=cp,
            )(g128, out_t)
    return jnp.transpose(out_t, (2, 0, 1))


# speedup vs baseline: 4.9127x; 1.0503x over previous
import functools
import math

import jax
import jax.numpy as jnp
from jax import lax
from jax.experimental import pallas as pl
from jax.experimental.pallas import tpu as pltpu
from jax.experimental.pallas import tpu_sc as plsc

_EMBED = 32
_SCALE = math.sqrt(float(_EMBED))
_NUM_WORKERS = 32
_CHUNK = 512
_PACK_C = 32768
_SLICES = 2
_PLANES_PER_STEP = 5


def _pack_body(in_ref, out_ref):
    a = in_ref[...] * _SCALE
    for j in range(_PACK_C // 512):
        cols = a[:, 512 * j:512 * (j + 1)]
        stacked = jnp.concatenate(
            [cols[:, 0:128], cols[:, 128:256],
             cols[:, 256:384], cols[:, 384:512]], axis=0)
        out_ref[128 * j:128 * (j + 1), :] = stacked.T


def _unpack_first_body(in_ref, out_ref):
    _unpack_planes(in_ref, out_ref)


def _unpack_body(in_ref, prev_ref, out_ref):
    del prev_ref
    _unpack_planes(in_ref, out_ref)


def _unpack_planes(in_ref, out_ref):
    for p in range(_PLANES_PER_STEP):
        for k in range(8):
            r = 1024 * p + 128 * k
            t = in_ref[r:r + 128, :].T
            for q in range(4):
                out_ref[p, :, 512 * k + 128 * q:512 * k + 128 * (q + 1)] = (
                    t[32 * q:32 * (q + 1), :])


def kernel(x, lut):
    batch, seq = x.shape
    n = batch * seq
    vocab, embed = lut.shape
    idx = jnp.transpose(x).reshape(n).astype(jnp.int32)

    lut_t = jnp.transpose(lut)
    grid = (vocab + _PACK_C - 1) // _PACK_C
    vocab_pad = grid * _PACK_C
    lut_packed = pl.pallas_call(
        _pack_body,
        grid=(grid,),
        in_specs=[pl.BlockSpec((embed, _PACK_C), lambda i: (0, i))],
        out_specs=pl.BlockSpec((_PACK_C * embed // 128, 128),
                               lambda i: (i, 0)),
        out_shape=jax.ShapeDtypeStruct((vocab_pad * embed // 128, 128),
                                       jnp.float32),
        compiler_params=pltpu.CompilerParams(
            dimension_semantics=("parallel",)),
    )(lut_t)
    lut_rows = lut_packed.reshape(vocab_pad, embed)

    n_sl = n // _SLICES
    seq_sl = seq // _SLICES
    per_w = n_sl // _NUM_WORKERS
    n_chunks = per_w // _CHUNK
    assert n_chunks * _CHUNK == per_w

    mesh = plsc.VectorSubcoreMesh(core_axis_name="c", subcore_axis_name="s")

    @functools.partial(
        pl.kernel,
        mesh=mesh,
        out_type=jax.ShapeDtypeStruct((n_sl, embed), jnp.float32),
        scratch_types=[
            pltpu.VMEM((_CHUNK,), jnp.int32),
            pltpu.VMEM((_CHUNK,), jnp.int32),
            pltpu.VMEM((_CHUNK,), jnp.int32),
            pltpu.VMEM((_CHUNK, embed), jnp.float32),
            pltpu.VMEM((_CHUNK, embed), jnp.float32),
            pltpu.SemaphoreType.DMA,
            pltpu.SemaphoreType.DMA,
            pltpu.SemaphoreType.DMA,
            pltpu.SemaphoreType.DMA,
        ],
        compiler_params=pltpu.CompilerParams(use_tc_tiling_on_sc=False,
                                             needs_layout_passes=False),
    )
    def sc_gather(idx_hbm, table_hbm, out_hbm,
                  idx_raw, idx0, idx1, rows0, rows1,
                  gsem0, gsem1, ssem0, ssem1):
        wid = lax.axis_index("s") * 2 + lax.axis_index("c")
        base = wid * per_w
        iota16 = lax.iota(jnp.int32, 16)

        def start(c, idx_b, rows_b, gsem_b):
            off = base + c * _CHUNK
            pltpu.sync_copy(idx_hbm.at[pl.ds(off, _CHUNK)], idx_raw)

            @pl.loop(0, _CHUNK, step=16)
            def _(m):
                v = idx_raw.at[pl.ds(m, 16)][...]
                rho = ((v & ~511) + ((v & 127) << 2) + ((v >> 7) & 3))
                pos = ((m & ~511) + ((m & 127) + iota16) * 4
                       + ((m >> 7) & 3))
                plsc.store_scatter(idx_b, [pos], rho)

            pltpu.async_copy(table_hbm.at[idx_b], rows_b, gsem_b)

        def gather_wait(idx_b, rows_b, gsem_b):
            pltpu.make_async_copy(table_hbm.at[idx_b], rows_b, gsem_b).wait()

        def store_wait(rows_b, ssem_b):
            pltpu.make_async_copy(
                rows_b, out_hbm.at[pl.ds(0, _CHUNK)], ssem_b).wait()

        start(0, idx0, rows0, gsem0)

        @pl.loop(0, (n_chunks // 2) * 2, step=2)
        def _(c):
            @pl.when(c > 0)
            def _():
                store_wait(rows1, ssem1)

            start(c + 1, idx1, rows1, gsem1)

            gather_wait(idx0, rows0, gsem0)
            pltpu.async_copy(
                rows0, out_hbm.at[pl.ds(base + c * _CHUNK, _CHUNK)], ssem0)

            @pl.when(c + 2 < n_chunks)
            def _():
                store_wait(rows0, ssem0)
                start(c + 2, idx0, rows0, gsem0)

            gather_wait(idx1, rows1, gsem1)
            pltpu.async_copy(
                rows1, out_hbm.at[pl.ds(base + (c + 1) * _CHUNK, _CHUNK)],
                ssem1)

        if n_chunks % 2 == 1:
            gather_wait(idx0, rows0, gsem0)
            pltpu.async_copy(
                rows0,
                out_hbm.at[pl.ds(base + (n_chunks - 1) * _CHUNK, _CHUNK)],
                ssem0)
        store_wait(rows1, ssem1)
        store_wait(rows0, ssem0)

    out_t = None
    for k in range(_SLICES):
        idx_k = lax.slice(idx, (k * n_sl,), ((k + 1) * n_sl,))
        gathered = sc_gather(idx_k, lut_rows)
        g128 = gathered.reshape(n_sl // 4, 128)
        out_map = lambda s, kk=k: (kk * (seq_sl // _PLANES_PER_STEP) + s,
                                   0, 0)
        in_spec = pl.BlockSpec((_PLANES_PER_STEP * batch // 4, 128),
                               lambda s: (s, 0))
        out_spec = pl.BlockSpec((_PLANES_PER_STEP, embed, batch), out_map)
        out_shape = jax.ShapeDtypeStruct((seq, embed, batch), jnp.float32)
        cp = pltpu.CompilerParams(dimension_semantics=("arbitrary",))
        if out_t is None:
            out_t = pl.pallas_call(
                _unpack_first_body,
                grid=(seq_sl // _PLANES_PER_STEP,),
                in_specs=[in_spec],
                out_specs=out_spec,
                out_shape=out_shape,
                compiler_params=cp,
            )(g128)
        else:
            out_t = pl.pallas_call(
                _unpack_body,
                grid=(seq_sl // _PLANES_PER_STEP,),
                in_specs=[in_spec,
                          pl.BlockSpec(memory_space=pl.ANY)],
                out_specs=out_spec,
                out_shape=out_shape,
                input_output_aliases={1: 0},
                compiler_params=cp,
            )(g128, out_t)
    return jnp.transpose(out_t, (2, 0, 1))


# speedup vs baseline: 4.9603x; 1.0097x over previous
import functools
import math

import jax
import jax.numpy as jnp
from jax import lax
from jax.experimental import pallas as pl
from jax.experimental.pallas import tpu as pltpu
from jax.experimental.pallas import tpu_sc as plsc

_EMBED = 32
_SCALE = math.sqrt(float(_EMBED))
_NUM_WORKERS = 32
_CHUNK = 512
_PACK_C = 32768
_SLICES = 5
_PLANES_PER_STEP = 5


def _pack_body(in_ref, out_ref):
    a = in_ref[...] * _SCALE
    for j in range(_PACK_C // 512):
        cols = a[:, 512 * j:512 * (j + 1)]
        stacked = jnp.concatenate(
            [cols[:, 0:128], cols[:, 128:256],
             cols[:, 256:384], cols[:, 384:512]], axis=0)
        out_ref[128 * j:128 * (j + 1), :] = stacked.T


def _unpack_first_body(in_ref, out_ref):
    _unpack_planes(in_ref, out_ref)


def _unpack_body(in_ref, prev_ref, out_ref):
    del prev_ref
    _unpack_planes(in_ref, out_ref)


def _unpack_planes(in_ref, out_ref):
    for p in range(_PLANES_PER_STEP):
        for k in range(8):
            r = 1024 * p + 128 * k
            t = in_ref[r:r + 128, :].T
            for q in range(4):
                out_ref[p, :, 512 * k + 128 * q:512 * k + 128 * (q + 1)] = (
                    t[32 * q:32 * (q + 1), :])


def kernel(x, lut):
    batch, seq = x.shape
    n = batch * seq
    vocab, embed = lut.shape
    idx = jnp.transpose(x).reshape(n).astype(jnp.int32)

    lut_t = jnp.transpose(lut)
    grid = (vocab + _PACK_C - 1) // _PACK_C
    vocab_pad = grid * _PACK_C
    lut_packed = pl.pallas_call(
        _pack_body,
        grid=(grid,),
        in_specs=[pl.BlockSpec((embed, _PACK_C), lambda i: (0, i))],
        out_specs=pl.BlockSpec((_PACK_C * embed // 128, 128),
                               lambda i: (i, 0)),
        out_shape=jax.ShapeDtypeStruct((vocab_pad * embed // 128, 128),
                                       jnp.float32),
        compiler_params=pltpu.CompilerParams(
            dimension_semantics=("parallel",)),
    )(lut_t)
    lut_rows = lut_packed.reshape(vocab_pad, embed)

    n_sl = n // _SLICES
    seq_sl = seq // _SLICES
    per_w = n_sl // _NUM_WORKERS
    n_chunks = per_w // _CHUNK
    assert n_chunks * _CHUNK == per_w

    mesh = plsc.VectorSubcoreMesh(core_axis_name="c", subcore_axis_name="s")

    @functools.partial(
        pl.kernel,
        mesh=mesh,
        out_type=jax.ShapeDtypeStruct((n_sl, embed), jnp.float32),
        scratch_types=[
            pltpu.VMEM((_CHUNK,), jnp.int32),
            pltpu.VMEM((_CHUNK,), jnp.int32),
            pltpu.VMEM((_CHUNK,), jnp.int32),
            pltpu.VMEM((_CHUNK, embed), jnp.float32),
            pltpu.VMEM((_CHUNK, embed), jnp.float32),
            pltpu.SemaphoreType.DMA,
            pltpu.SemaphoreType.DMA,
            pltpu.SemaphoreType.DMA,
            pltpu.SemaphoreType.DMA,
        ],
        compiler_params=pltpu.CompilerParams(use_tc_tiling_on_sc=False,
                                             needs_layout_passes=False),
    )
    def sc_gather(idx_hbm, table_hbm, out_hbm,
                  idx_raw, idx0, idx1, rows0, rows1,
                  gsem0, gsem1, ssem0, ssem1):
        wid = lax.axis_index("s") * 2 + lax.axis_index("c")
        base = wid * per_w
        iota16 = lax.iota(jnp.int32, 16)

        def start(c, idx_b, rows_b, gsem_b):
            off = base + c * _CHUNK
            pltpu.sync_copy(idx_hbm.at[pl.ds(off, _CHUNK)], idx_raw)

            @pl.loop(0, _CHUNK, step=16)
            def _(m):
                v = idx_raw.at[pl.ds(m, 16)][...]
                rho = ((v & ~511) + ((v & 127) << 2) + ((v >> 7) & 3))
                pos = ((m & ~511) + ((m & 127) + iota16) * 4
                       + ((m >> 7) & 3))
                plsc.store_scatter(idx_b, [pos], rho)

            pltpu.async_copy(table_hbm.at[idx_b], rows_b, gsem_b)

        def gather_wait(idx_b, rows_b, gsem_b):
            pltpu.make_async_copy(table_hbm.at[idx_b], rows_b, gsem_b).wait()

        def store_wait(rows_b, ssem_b):
            pltpu.make_async_copy(
                rows_b, out_hbm.at[pl.ds(0, _CHUNK)], ssem_b).wait()

        start(0, idx0, rows0, gsem0)

        @pl.loop(0, (n_chunks // 2) * 2, step=2)
        def _(c):
            @pl.when(c > 0)
            def _():
                store_wait(rows1, ssem1)

            start(c + 1, idx1, rows1, gsem1)

            gather_wait(idx0, rows0, gsem0)
            pltpu.async_copy(
                rows0, out_hbm.at[pl.ds(base + c * _CHUNK, _CHUNK)], ssem0)

            @pl.when(c + 2 < n_chunks)
            def _():
                store_wait(rows0, ssem0)
                start(c + 2, idx0, rows0, gsem0)

            gather_wait(idx1, rows1, gsem1)
            pltpu.async_copy(
                rows1, out_hbm.at[pl.ds(base + (c + 1) * _CHUNK, _CHUNK)],
                ssem1)

        if n_chunks % 2 == 1:
            gather_wait(idx0, rows0, gsem0)
            pltpu.async_copy(
                rows0,
                out_hbm.at[pl.ds(base + (n_chunks - 1) * _CHUNK, _CHUNK)],
                ssem0)
        store_wait(rows1, ssem1)
        store_wait(rows0, ssem0)

    out_t = None
    for k in range(_SLICES):
        idx_k = lax.slice(idx, (k * n_sl,), ((k + 1) * n_sl,))
        gathered = sc_gather(idx_k, lut_rows)
        g128 = gathered.reshape(n_sl // 4, 128)
        out_map = lambda s, kk=k: (kk * (seq_sl // _PLANES_PER_STEP) + s,
                                   0, 0)
        in_spec = pl.BlockSpec((_PLANES_PER_STEP * batch // 4, 128),
                               lambda s: (s, 0))
        out_spec = pl.BlockSpec((_PLANES_PER_STEP, embed, batch), out_map)
        out_shape = jax.ShapeDtypeStruct((seq, embed, batch), jnp.float32)
        cp = pltpu.CompilerParams(dimension_semantics=("arbitrary",))
        if out_t is None:
            out_t = pl.pallas_call(
                _unpack_first_body,
                grid=(seq_sl // _PLANES_PER_STEP,),
                in_specs=[in_spec],
                out_specs=out_spec,
                out_shape=out_shape,
                compiler_params=cp,
            )(g128)
        else:
            out_t = pl.pallas_call(
                _unpack_body,
                grid=(seq_sl // _PLANES_PER_STEP,),
                in_specs=[in_spec,
                          pl.BlockSpec(memory_space=pl.ANY)],
                out_specs=out_spec,
                out_shape=out_shape,
                input_output_aliases={1: 0},
                compiler_params=cp,
            )(g128, out_t)
    return jnp.transpose(out_t, (2, 0, 1))


# speedup vs baseline: 5.0977x; 1.0277x over previous
import functools
import math

import jax
import jax.numpy as jnp
from jax import lax
from jax.experimental import pallas as pl
from jax.experimental.pallas import tpu as pltpu
from jax.experimental.pallas import tpu_sc as plsc

_EMBED = 32
_SCALE = math.sqrt(float(_EMBED))
_NUM_WORKERS = 32
_CHUNK = 1024
_PACK_C = 32768
_SLICES = 5
_PLANES_PER_STEP = 8


def _pack_body(in_ref, out_ref):
    a = in_ref[...] * _SCALE
    for j in range(_PACK_C // 512):
        cols = a[:, 512 * j:512 * (j + 1)]
        stacked = jnp.concatenate(
            [cols[:, 0:128], cols[:, 128:256],
             cols[:, 256:384], cols[:, 384:512]], axis=0)
        out_ref[128 * j:128 * (j + 1), :] = stacked.T


def _unpack_first_body(in_ref, out_ref):
    _unpack_planes(in_ref, out_ref)


def _unpack_body(in_ref, prev_ref, out_ref):
    del prev_ref
    _unpack_planes(in_ref, out_ref)


def _unpack_planes(in_ref, out_ref):
    for p in range(_PLANES_PER_STEP):
        for k in range(8):
            r = 1024 * p + 128 * k
            t = in_ref[r:r + 128, :].T
            for q in range(4):
                out_ref[p, :, 512 * k + 128 * q:512 * k + 128 * (q + 1)] = (
                    t[32 * q:32 * (q + 1), :])


def kernel(x, lut):
    batch, seq = x.shape
    n = batch * seq
    vocab, embed = lut.shape
    idx = jnp.transpose(x).reshape(n).astype(jnp.int32)

    lut_t = jnp.transpose(lut)
    grid = (vocab + _PACK_C - 1) // _PACK_C
    vocab_pad = grid * _PACK_C
    lut_packed = pl.pallas_call(
        _pack_body,
        grid=(grid,),
        in_specs=[pl.BlockSpec((embed, _PACK_C), lambda i: (0, i))],
        out_specs=pl.BlockSpec((_PACK_C * embed // 128, 128),
                               lambda i: (i, 0)),
        out_shape=jax.ShapeDtypeStruct((vocab_pad * embed // 128, 128),
                                       jnp.float32),
        compiler_params=pltpu.CompilerParams(
            dimension_semantics=("parallel",)),
    )(lut_t)
    lut_rows = lut_packed.reshape(vocab_pad, embed)

    n_sl = n // _SLICES
    seq_sl = seq // _SLICES
    per_w = n_sl // _NUM_WORKERS
    n_chunks = per_w // _CHUNK
    assert n_chunks * _CHUNK == per_w

    mesh = plsc.VectorSubcoreMesh(core_axis_name="c", subcore_axis_name="s")

    @functools.partial(
        pl.kernel,
        mesh=mesh,
        out_type=jax.ShapeDtypeStruct((n_sl, embed), jnp.float32),
        scratch_types=[
            pltpu.VMEM((_CHUNK,), jnp.int32),
            pltpu.VMEM((_CHUNK,), jnp.int32),
            pltpu.VMEM((_CHUNK,), jnp.int32),
            pltpu.VMEM((_CHUNK, embed), jnp.float32),
            pltpu.VMEM((_CHUNK, embed), jnp.float32),
            pltpu.SemaphoreType.DMA,
            pltpu.SemaphoreType.DMA,
            pltpu.SemaphoreType.DMA,
            pltpu.SemaphoreType.DMA,
        ],
        compiler_params=pltpu.CompilerParams(use_tc_tiling_on_sc=False,
                                             needs_layout_passes=False),
    )
    def sc_gather(idx_hbm, table_hbm, out_hbm,
                  idx_raw, idx0, idx1, rows0, rows1,
                  gsem0, gsem1, ssem0, ssem1):
        wid = lax.axis_index("s") * 2 + lax.axis_index("c")
        base = wid * per_w
        iota16 = lax.iota(jnp.int32, 16)

        def start(c, idx_b, rows_b, gsem_b):
            off = base + c * _CHUNK
            pltpu.sync_copy(idx_hbm.at[pl.ds(off, _CHUNK)], idx_raw)

            @pl.loop(0, _CHUNK, step=16)
            def _(m):
                v = idx_raw.at[pl.ds(m, 16)][...]
                rho = ((v & ~511) + ((v & 127) << 2) + ((v >> 7) & 3))
                pos = ((m & ~511) + ((m & 127) + iota16) * 4
                       + ((m >> 7) & 3))
                plsc.store_scatter(idx_b, [pos], rho)

            pltpu.async_copy(table_hbm.at[idx_b], rows_b, gsem_b)

        def gather_wait(idx_b, rows_b, gsem_b):
            pltpu.make_async_copy(table_hbm.at[idx_b], rows_b, gsem_b).wait()

        def store_wait(rows_b, ssem_b):
            pltpu.make_async_copy(
                rows_b, out_hbm.at[pl.ds(0, _CHUNK)], ssem_b).wait()

        start(0, idx0, rows0, gsem0)

        @pl.loop(0, (n_chunks // 2) * 2, step=2)
        def _(c):
            @pl.when(c > 0)
            def _():
                store_wait(rows1, ssem1)

            start(c + 1, idx1, rows1, gsem1)

            gather_wait(idx0, rows0, gsem0)
            pltpu.async_copy(
                rows0, out_hbm.at[pl.ds(base + c * _CHUNK, _CHUNK)], ssem0)

            @pl.when(c + 2 < n_chunks)
            def _():
                store_wait(rows0, ssem0)
                start(c + 2, idx0, rows0, gsem0)

            gather_wait(idx1, rows1, gsem1)
            pltpu.async_copy(
                rows1, out_hbm.at[pl.ds(base + (c + 1) * _CHUNK, _CHUNK)],
                ssem1)

        if n_chunks % 2 == 1:
            gather_wait(idx0, rows0, gsem0)
            pltpu.async_copy(
                rows0,
                out_hbm.at[pl.ds(base + (n_chunks - 1) * _CHUNK, _CHUNK)],
                ssem0)
        store_wait(rows1, ssem1)
        store_wait(rows0, ssem0)

    out_t = None
    for k in range(_SLICES):
        idx_k = lax.slice(idx, (k * n_sl,), ((k + 1) * n_sl,))
        gathered = sc_gather(idx_k, lut_rows)
        g128 = gathered.reshape(n_sl // 4, 128)
        out_map = lambda s, kk=k: (kk * (seq_sl // _PLANES_PER_STEP) + s,
                                   0, 0)
        in_spec = pl.BlockSpec((_PLANES_PER_STEP * batch // 4, 128),
                               lambda s: (s, 0))
        out_spec = pl.BlockSpec((_PLANES_PER_STEP, embed, batch), out_map)
        out_shape = jax.ShapeDtypeStruct((seq, embed, batch), jnp.float32)
        cp = pltpu.CompilerParams(dimension_semantics=("arbitrary",))
        if out_t is None:
            out_t = pl.pallas_call(
                _unpack_first_body,
                grid=(seq_sl // _PLANES_PER_STEP,),
                in_specs=[in_spec],
                out_specs=out_spec,
                out_shape=out_shape,
                compiler_params=cp,
            )(g128)
        else:
            out_t = pl.pallas_call(
                _unpack_body,
                grid=(seq_sl // _PLANES_PER_STEP,),
                in_specs=[in_spec,
                          pl.BlockSpec(memory_space=pl.ANY)],
                out_specs=out_spec,
                out_shape=out_shape,
                input_output_aliases={1: 0},
                compiler_params=cp,
            )(g128, out_t)
    return jnp.transpose(out_t, (2, 0, 1))


# speedup vs baseline: 5.1211x; 1.0046x over previous
import functools
import math

import jax
import jax.numpy as jnp
from jax import lax
from jax.experimental import pallas as pl
from jax.experimental.pallas import tpu as pltpu
from jax.experimental.pallas import tpu_sc as plsc

_EMBED = 32
_SCALE = math.sqrt(float(_EMBED))
_NUM_WORKERS = 32
_CHUNK = 1024
_PACK_C = 65536
_SLICES = 5
_PLANES_PER_STEP = 8


def _pack_body(in_ref, out_ref):
    a = in_ref[...] * _SCALE
    for j in range(_PACK_C // 512):
        cols = a[:, 512 * j:512 * (j + 1)]
        stacked = jnp.concatenate(
            [cols[:, 0:128], cols[:, 128:256],
             cols[:, 256:384], cols[:, 384:512]], axis=0)
        out_ref[128 * j:128 * (j + 1), :] = stacked.T


def _unpack_first_body(in_ref, out_ref):
    _unpack_planes(in_ref, out_ref)


def _unpack_body(in_ref, prev_ref, out_ref):
    del prev_ref
    _unpack_planes(in_ref, out_ref)


def _unpack_planes(in_ref, out_ref):
    for p in range(_PLANES_PER_STEP):
        for k in range(8):
            r = 1024 * p + 128 * k
            t = in_ref[r:r + 128, :].T
            for q in range(4):
                out_ref[p, :, 512 * k + 128 * q:512 * k + 128 * (q + 1)] = (
                    t[32 * q:32 * (q + 1), :])


def kernel(x, lut):
    batch, seq = x.shape
    n = batch * seq
    vocab, embed = lut.shape
    idx = jnp.transpose(x).reshape(n).astype(jnp.int32)

    lut_t = jnp.transpose(lut)
    grid = (vocab + _PACK_C - 1) // _PACK_C
    vocab_pad = grid * _PACK_C
    lut_packed = pl.pallas_call(
        _pack_body,
        grid=(grid,),
        in_specs=[pl.BlockSpec((embed, _PACK_C), lambda i: (0, i))],
        out_specs=pl.BlockSpec((_PACK_C * embed // 128, 128),
                               lambda i: (i, 0)),
        out_shape=jax.ShapeDtypeStruct((vocab_pad * embed // 128, 128),
                                       jnp.float32),
        compiler_params=pltpu.CompilerParams(
            dimension_semantics=("parallel",)),
    )(lut_t)
    lut_rows = lut_packed.reshape(vocab_pad, embed)

    n_sl = n // _SLICES
    seq_sl = seq // _SLICES
    per_w = n_sl // _NUM_WORKERS
    n_chunks = per_w // _CHUNK
    assert n_chunks * _CHUNK == per_w

    mesh = plsc.VectorSubcoreMesh(core_axis_name="c", subcore_axis_name="s")

    @functools.partial(
        pl.kernel,
        mesh=mesh,
        out_type=jax.ShapeDtypeStruct((n_sl, embed), jnp.float32),
        scratch_types=[
            pltpu.VMEM((_CHUNK,), jnp.int32),
            pltpu.VMEM((_CHUNK,), jnp.int32),
            pltpu.VMEM((_CHUNK,), jnp.int32),
            pltpu.VMEM((_CHUNK, embed), jnp.float32),
            pltpu.VMEM((_CHUNK, embed), jnp.float32),
            pltpu.SemaphoreType.DMA,
            pltpu.SemaphoreType.DMA,
            pltpu.SemaphoreType.DMA,
            pltpu.SemaphoreType.DMA,
        ],
        compiler_params=pltpu.CompilerParams(use_tc_tiling_on_sc=False,
                                             needs_layout_passes=False),
    )
    def sc_gather(idx_hbm, table_hbm, out_hbm,
                  idx_raw, idx0, idx1, rows0, rows1,
                  gsem0, gsem1, ssem0, ssem1):
        wid = lax.axis_index("s") * 2 + lax.axis_index("c")
        base = wid * per_w
        iota16 = lax.iota(jnp.int32, 16)

        def start(c, idx_b, rows_b, gsem_b):
            off = base + c * _CHUNK
            pltpu.sync_copy(idx_hbm.at[pl.ds(off, _CHUNK)], idx_raw)

            @pl.loop(0, _CHUNK, step=16)
            def _(m):
                v = idx_raw.at[pl.ds(m, 16)][...]
                rho = ((v & ~511) + ((v & 127) << 2) + ((v >> 7) & 3))
                pos = ((m & ~511) + ((m & 127) + iota16) * 4
                       + ((m >> 7) & 3))
                plsc.store_scatter(idx_b, [pos], rho)

            pltpu.async_copy(table_hbm.at[idx_b], rows_b, gsem_b)

        def gather_wait(idx_b, rows_b, gsem_b):
            pltpu.make_async_copy(table_hbm.at[idx_b], rows_b, gsem_b).wait()

        def store_wait(rows_b, ssem_b):
            pltpu.make_async_copy(
                rows_b, out_hbm.at[pl.ds(0, _CHUNK)], ssem_b).wait()

        start(0, idx0, rows0, gsem0)

        @pl.loop(0, (n_chunks // 2) * 2, step=2)
        def _(c):
            @pl.when(c > 0)
            def _():
                store_wait(rows1, ssem1)

            start(c + 1, idx1, rows1, gsem1)

            gather_wait(idx0, rows0, gsem0)
            pltpu.async_copy(
                rows0, out_hbm.at[pl.ds(base + c * _CHUNK, _CHUNK)], ssem0)

            @pl.when(c + 2 < n_chunks)
            def _():
                store_wait(rows0, ssem0)
                start(c + 2, idx0, rows0, gsem0)

            gather_wait(idx1, rows1, gsem1)
            pltpu.async_copy(
                rows1, out_hbm.at[pl.ds(base + (c + 1) * _CHUNK, _CHUNK)],
                ssem1)

        if n_chunks % 2 == 1:
            gather_wait(idx0, rows0, gsem0)
            pltpu.async_copy(
                rows0,
                out_hbm.at[pl.ds(base + (n_chunks - 1) * _CHUNK, _CHUNK)],
                ssem0)
        store_wait(rows1, ssem1)
        store_wait(rows0, ssem0)

    out_t = None
    for k in range(_SLICES):
        idx_k = lax.slice(idx, (k * n_sl,), ((k + 1) * n_sl,))
        gathered = sc_gather(idx_k, lut_rows)
        g128 = gathered.reshape(n_sl // 4, 128)
        out_map = lambda s, kk=k: (kk * (seq_sl // _PLANES_PER_STEP) + s,
                                   0, 0)
        in_spec = pl.BlockSpec((_PLANES_PER_STEP * batch // 4, 128),
                               lambda s: (s, 0))
        out_spec = pl.BlockSpec((_PLANES_PER_STEP, embed, batch), out_map)
        out_shape = jax.ShapeDtypeStruct((seq, embed, batch), jnp.float32)
        cp = pltpu.CompilerParams(dimension_semantics=("arbitrary",))
        if out_t is None:
            out_t = pl.pallas_call(
                _unpack_first_body,
                grid=(seq_sl // _PLANES_PER_STEP,),
                in_specs=[in_spec],
                out_specs=out_spec,
                out_shape=out_shape,
                compiler_params=cp,
            )(g128)
        else:
            out_t = pl.pallas_call(
                _unpack_body,
                grid=(seq_sl // _PLANES_PER_STEP,),
                in_specs=[in_spec,
                          pl.BlockSpec(memory_space=pl.ANY)],
                out_specs=out_spec,
                out_shape=out_shape,
                input_output_aliases={1: 0},
                compiler_params=cp,
            )(g128, out_t)
    return jnp.transpose(out_t, (2, 0, 1))
